# baseline (speedup 1.0000x reference)
import jax
import jax.numpy as jnp
from jax import lax
from jax.experimental import pallas as pl
from jax.experimental.pallas import tpu as pltpu
from jax.experimental.pallas import tpu_sc as plsc

BATCH = 4096
MAXLEN = 200
EMBED_DIM = 64

NW = 32
CHUNKS = MAXLEN * (EMBED_DIM // 8)
PER_W = CHUNKS // NW


def _sc_body(x_hbm, out_hbm, xbuf0, xbuf1):
    wid = lax.axis_index("s") * 2 + lax.axis_index("c")
    base = wid * PER_W

    def chunk(i, carry):
        c = base + i
        pltpu.sync_copy(x_hbm.at[c], xbuf0)
        pltpu.sync_copy(xbuf0, out_hbm.at[c])
        return carry

    lax.fori_loop(0, PER_W, chunk, 0)
    del xbuf1


def kernel(x, pos_table):
    xt = x.transpose(1, 2, 0)
    x5 = xt.reshape(MAXLEN, 8, 8, 32, 128).transpose(0, 1, 3, 2, 4)
    x5 = x5.reshape(CHUNKS, 32, 8, 128)
    mesh = plsc.VectorSubcoreMesh(core_axis_name="c", subcore_axis_name="s")
    out5 = pl.kernel(
        _sc_body,
        out_type=jax.ShapeDtypeStruct((CHUNKS, 32, 8, 128), jnp.float32),
        mesh=mesh,
        scratch_types=[
            pltpu.VMEM((32, 8, 128), jnp.float32),
            pltpu.VMEM((32, 8, 128), jnp.float32),
        ],
    )(x5)
    out_t = (
        out5.reshape(MAXLEN, 8, 32, 8, 128)
        .transpose(0, 1, 3, 2, 4)
        .reshape(MAXLEN, EMBED_DIM, BATCH)
    )
    return out_t.transpose(2, 0, 1)


# speedup vs baseline: 1.1213x; 1.1213x over previous
import jax
import jax.numpy as jnp
from jax import lax
from jax.experimental import pallas as pl
from jax.experimental.pallas import tpu as pltpu
from jax.experimental.pallas import tpu_sc as plsc

BATCH = 4096
MAXLEN = 200
EMBED_DIM = 64

NW = 32
CHUNKS = MAXLEN * (EMBED_DIM // 8)
PER_W = CHUNKS // NW
HALF = 16


def _sc_body(x_hbm, pos_hbm, out_hbm, i0, i1, o0, o1, pbuf, si0, si1, so0, so1):
    ibuf = (i0, i1)
    obuf = (o0, o1)
    sin = (si0, si1)
    sout = (so0, so1)
    wid = lax.axis_index("s") * 2 + lax.axis_index("c")
    base = wid * PER_W
    pltpu.sync_copy(pos_hbm.at[pl.ds(base, PER_W)], pbuf)
    for h in range(2):
        pltpu.async_copy(
            x_hbm.at[base, pl.ds(h * HALF, HALF)], ibuf[h], sin[h]
        )

    def chunk(cc, carry):
        c = base + cc
        for h in range(2):
            hs = pl.ds(h * HALF, HALF)
            pltpu.make_async_copy(x_hbm.at[c, hs], ibuf[h], sin[h]).wait()

            @pl.when(cc > 0)
            def _():
                pltpu.make_async_copy(obuf[h], out_hbm.at[c - 1, hs], sout[h]).wait()

            for e in range(8):
                pv = pbuf[cc, e, :]

                def tile(t, carry2):
                    for g in range(8):
                        sl = pl.ds(g * 16, 16)
                        obuf[h][t, e, sl] = ibuf[h][t, e, sl] + pv
                    return carry2

                lax.fori_loop(0, HALF, tile, 0)
            pltpu.async_copy(obuf[h], out_hbm.at[c, hs], sout[h])

            @pl.when(cc + 1 < PER_W)
            def _():
                pltpu.async_copy(
                    x_hbm.at[c + 1, hs], ibuf[h], sin[h]
                )

        return carry

    lax.fori_loop(0, PER_W, chunk, 0)
    for h in range(2):
        hs = pl.ds(h * HALF, HALF)
        pltpu.make_async_copy(
            obuf[h], out_hbm.at[base + PER_W - 1, hs], sout[h]
        ).wait()


def _sc_add(x5, pos_splat):
    mesh = plsc.VectorSubcoreMesh(core_axis_name="c", subcore_axis_name="s")
    return pl.kernel(
        _sc_body,
        out_type=jax.ShapeDtypeStruct((CHUNKS, 32, 8, 128), jnp.float32),
        mesh=mesh,
        scratch_types=[
            pltpu.VMEM((HALF, 8, 128), jnp.float32),
            pltpu.VMEM((HALF, 8, 128), jnp.float32),
            pltpu.VMEM((HALF, 8, 128), jnp.float32),
            pltpu.VMEM((HALF, 8, 128), jnp.float32),
            pltpu.VMEM((PER_W, 8, 16), jnp.float32),
            pltpu.SemaphoreType.DMA,
            pltpu.SemaphoreType.DMA,
            pltpu.SemaphoreType.DMA,
            pltpu.SemaphoreType.DMA,
        ],
    )(x5, pos_splat)


def kernel(x, pos_table):
    xt = x.transpose(1, 2, 0)
    x5 = xt.reshape(MAXLEN, 8, 8, 32, 128).transpose(0, 1, 3, 2, 4)
    x5 = x5.reshape(CHUNKS, 32, 8, 128)
    pos_splat = jnp.broadcast_to(
        pos_table.reshape(CHUNKS, 8)[:, :, None], (CHUNKS, 8, 16)
    )
    out5 = _sc_add(x5, pos_splat)
    out_t = (
        out5.reshape(MAXLEN, 8, 32, 8, 128)
        .transpose(0, 1, 3, 2, 4)
        .reshape(MAXLEN, EMBED_DIM, BATCH)
    )
    return out_t.transpose(2, 0, 1)
